# Initial kernel scaffold; baseline (speedup 1.0000x reference)
import functools
import math

import jax
import jax.numpy as jnp
import numpy as np
from jax.experimental import pallas as pl
from jax.experimental.pallas import tpu as pltpu

N = 10000
E = 160000
F_IN = 256
HEADS = 8
C1 = 128
C2 = 8
D1 = HEADS * C1
D2 = HEADS * C2
NUM_GRAPHS = 16
NB = 400
NBLK = N // NB

_DOT = functools.partial(jax.lax.dot_general,
                         dimension_numbers=(((1,), (0,)), ((), ())),
                         preferred_element_type=jnp.float32,
                         precision=jax.lax.Precision.HIGHEST)


def _qkvs1_body(x_ref, wq_ref, wk_ref, wv_ref, ws_ref, bq_ref, bk_ref, bv_ref,
                bs_ref, q_ref, k_ref, v_ref, s_ref):
    x = x_ref[...]
    q_ref[0] = _DOT(x, wq_ref[...]) + bq_ref[0][None, :]
    k_ref[0] = _DOT(x, wk_ref[...]) + bk_ref[0][None, :]
    v_ref[0] = _DOT(x, wv_ref[...]) + bv_ref[0][None, :]
    s_ref[...] = _DOT(x, ws_ref[...]) + bs_ref[0][None, :]


def tc_qkvs_l1(x, cp):
    wq, wk, wv, ws = cp["q"]["W"], cp["k"]["W"], cp["v"]["W"], cp["skip"]["W"]
    bq = cp["q"]["b"].reshape(HEADS, C1)
    bk = cp["k"]["b"].reshape(HEADS, C1)
    bv = cp["v"]["b"].reshape(HEADS, C1)
    bs = cp["skip"]["b"].reshape(HEADS, C1)
    grid = (HEADS, NBLK)
    q, k, v, s = pl.pallas_call(
        _qkvs1_body,
        grid=grid,
        in_specs=[
            pl.BlockSpec((NB, F_IN), lambda h, nb: (nb, 0)),
            pl.BlockSpec((F_IN, C1), lambda h, nb: (0, h)),
            pl.BlockSpec((F_IN, C1), lambda h, nb: (0, h)),
            pl.BlockSpec((F_IN, C1), lambda h, nb: (0, h)),
            pl.BlockSpec((F_IN, C1), lambda h, nb: (0, h)),
            pl.BlockSpec((1, C1), lambda h, nb: (h, 0)),
            pl.BlockSpec((1, C1), lambda h, nb: (h, 0)),
            pl.BlockSpec((1, C1), lambda h, nb: (h, 0)),
            pl.BlockSpec((1, C1), lambda h, nb: (h, 0)),
        ],
        out_specs=[
            pl.BlockSpec((1, NB, C1), lambda h, nb: (h, nb, 0)),
            pl.BlockSpec((1, NB, C1), lambda h, nb: (h, nb, 0)),
            pl.BlockSpec((1, NB, C1), lambda h, nb: (h, nb, 0)),
            pl.BlockSpec((NB, C1), lambda h, nb: (nb, h)),
        ],
        out_shape=[
            jax.ShapeDtypeStruct((HEADS, N, C1), jnp.float32),
            jax.ShapeDtypeStruct((HEADS, N, C1), jnp.float32),
            jax.ShapeDtypeStruct((HEADS, N, C1), jnp.float32),
            jax.ShapeDtypeStruct((N, D1), jnp.float32),
        ],
    )(x, wq, wk, wv, ws, bq, bk, bv, bs)
    return q, k, v, s


def _qkvs2_body(x_ref, wq_ref, wk_ref, wv_ref, ws_ref, b_ref, q_ref, k_ref,
                v_ref, s_ref):
    x = x_ref[...]
    q_ref[...] = _DOT(x, wq_ref[...]) + b_ref[0][None, :]
    k_ref[...] = _DOT(x, wk_ref[...]) + b_ref[1][None, :]
    v_ref[...] = _DOT(x, wv_ref[...]) + b_ref[2][None, :]
    s_ref[...] = _DOT(x, ws_ref[...]) + b_ref[3][None, :]


def tc_qkvs_l2(h1, cp, perm):
    wq = cp["q"]["W"][:, perm]
    wk = cp["k"]["W"][:, perm]
    wv = cp["v"]["W"]
    ws = cp["skip"]["W"]
    b = jnp.stack([cp["q"]["b"][perm], cp["k"]["b"][perm], cp["v"]["b"],
                   cp["skip"]["b"]])
    outs = pl.pallas_call(
        _qkvs2_body,
        grid=(NBLK,),
        in_specs=[
            pl.BlockSpec((NB, D1), lambda nb: (nb, 0)),
            pl.BlockSpec((D1, D2), lambda nb: (0, 0)),
            pl.BlockSpec((D1, D2), lambda nb: (0, 0)),
            pl.BlockSpec((D1, D2), lambda nb: (0, 0)),
            pl.BlockSpec((D1, D2), lambda nb: (0, 0)),
            pl.BlockSpec((4, D2), lambda nb: (0, 0)),
        ],
        out_specs=[pl.BlockSpec((NB, D2), lambda nb: (nb, 0))] * 4,
        out_shape=[jax.ShapeDtypeStruct((N, D2), jnp.float32)] * 4,
    )(h1, wq, wk, wv, ws, b)
    return outs


def _combine1_body(on_ref, dn_ref, sk_ref, h_ref):
    den = dn_ref[0]
    h_ref[...] = jnp.maximum(on_ref[0] / (den + 1e-16) + sk_ref[...], 0.0)


def tc_combine_l1(outnum, denom, skip):
    return pl.pallas_call(
        _combine1_body,
        grid=(HEADS, NBLK),
        in_specs=[
            pl.BlockSpec((1, NB, C1), lambda h, nb: (h, nb, 0)),
            pl.BlockSpec((1, NB, 1), lambda h, nb: (h, nb, 0)),
            pl.BlockSpec((NB, C1), lambda h, nb: (nb, h)),
        ],
        out_specs=pl.BlockSpec((NB, C1), lambda h, nb: (nb, h)),
        out_shape=jax.ShapeDtypeStruct((N, D1), jnp.float32),
    )(outnum, denom, skip)


def _combine2_body(on_ref, dn_ref, sk_ref, h_ref):
    h_ref[...] = jnp.maximum(
        on_ref[...] / (dn_ref[...] + 1e-16) + sk_ref[...], 0.0)


def tc_combine_l2(outnum2, den2, skip2):
    return pl.pallas_call(
        _combine2_body,
        grid=(NBLK,),
        in_specs=[pl.BlockSpec((NB, D2), lambda nb: (nb, 0))] * 3,
        out_specs=pl.BlockSpec((NB, D2), lambda nb: (nb, 0)),
        out_shape=jax.ShapeDtypeStruct((N, D2), jnp.float32),
    )(outnum2, den2, skip2)


def _pool_body(b_ref, h_ref, p_ref):
    nb = pl.program_id(0)

    @pl.when(nb == 0)
    def _():
        p_ref[...] = jnp.full((NUM_GRAPHS, D2), -jnp.inf, jnp.float32)

    b = b_ref[0]
    blk = h_ref[...]
    for g in range(NUM_GRAPHS):
        m = jnp.where(b == g, blk, -jnp.inf)
        p_ref[g, :] = jnp.maximum(p_ref[g, :], jnp.max(m, axis=0))


def tc_pool(h2, batch):
    b3 = batch.reshape(NBLK, NB, 1)
    return pl.pallas_call(
        _pool_body,
        grid=(NBLK,),
        in_specs=[
            pl.BlockSpec((1, NB, 1), lambda nb: (nb, 0, 0)),
            pl.BlockSpec((NB, D2), lambda nb: (nb, 0)),
        ],
        out_specs=pl.BlockSpec((NUM_GRAPHS, D2), lambda nb: (0, 0)),
        out_shape=jax.ShapeDtypeStruct((NUM_GRAPHS, D2), jnp.float32),
    )(b3, h2)


def _head_body(p5_ref, p10_ref, p20_ref, w5_ref, w10_ref, w20_ref, b3_ref,
               w1_ref, b1_ref, w2_ref, b2_ref, o_ref):
    outs = []
    for i, pref in enumerate((p5_ref, p10_ref, p20_ref)):
        p = pref[...]
        p = jnp.where(jnp.isfinite(p), p, 0.0)
        w = (w5_ref, w10_ref, w20_ref)[i][...]
        outs.append(_DOT(p, w) + b3_ref[i][None, :])
    x = jnp.concatenate(outs, axis=1)
    h = jnp.maximum(_DOT(x, w1_ref[...]) + b1_ref[0][None, :], 0.0)
    o_ref[...] = _DOT(h, w2_ref[...]) + b2_ref[0][None, :]


def tc_head(p5, p10, p20, params):
    b3 = jnp.stack([params["b5"]["mlp"]["b"], params["b10"]["mlp"]["b"],
                    params["b20"]["mlp"]["b"]])
    full = lambda *shape: pl.BlockSpec(shape, lambda: tuple(0 for _ in shape))
    return pl.pallas_call(
        _head_body,
        in_specs=[
            full(NUM_GRAPHS, D2), full(NUM_GRAPHS, D2), full(NUM_GRAPHS, D2),
            full(D2, 8), full(D2, 8), full(D2, 8), full(3, 8),
            full(24, 8), full(1, 8), full(8, 10), full(1, 10),
        ],
        out_specs=full(NUM_GRAPHS, 10),
        out_shape=jax.ShapeDtypeStruct((NUM_GRAPHS, 10), jnp.float32),
    )(p5, p10, p20, params["b5"]["mlp"]["W"], params["b10"]["mlp"]["W"],
      params["b20"]["mlp"]["W"], b3, params["fc1"]["W"],
      params["fc1"]["b"].reshape(1, 8), params["fc2"]["W"],
      params["fc2"]["b"].reshape(1, 10))


def _edge_l1(q, k, v, src, dst):
    qi = q[:, dst]
    kj = k[:, src]
    alpha = jnp.sum(qi * kj, axis=-1) / np.sqrt(float(C1))
    g = jnp.max(alpha, axis=1, keepdims=True)
    ex = jnp.exp(alpha - g)
    denom = jax.vmap(lambda e: jax.ops.segment_sum(e, dst, num_segments=N))(ex)
    vj = v[:, src]
    outnum = jax.vmap(lambda e, vv: jax.ops.segment_sum(e[:, None] * vv, dst,
                                                        num_segments=N))(ex, vj)
    return outnum, denom[..., None]


def _edge_l2(q2t, k2t, v2, src, dst):
    qi = q2t[dst].reshape(E, C2, HEADS)
    kj = k2t[src].reshape(E, C2, HEADS)
    alpha = jnp.sum(qi * kj, axis=1) / np.sqrt(float(C2))
    g = jnp.max(alpha, axis=0, keepdims=True)
    ex = jnp.exp(alpha - g)
    exr = jnp.repeat(ex, C2, axis=1)
    vj = v2[src]
    outnum = jax.ops.segment_sum(exr * vj, dst, num_segments=N)
    den = jax.ops.segment_sum(exr, dst, num_segments=N)
    return outnum, den


_PERM = np.arange(D2).reshape(HEADS, C2).T.reshape(-1)


def _branch(x, edge_index, batch, bp):
    src, dst = edge_index[0], edge_index[1]
    q, k, v, skip = tc_qkvs_l1(x, bp["c1"])
    outnum, denom = _edge_l1(q, k, v, src, dst)
    h1 = tc_combine_l1(outnum, denom, skip)
    q2t, k2t, v2, skip2 = tc_qkvs_l2(h1, bp["c2"], _PERM)
    outnum2, den2 = _edge_l2(q2t, k2t, v2, src, dst)
    h2 = tc_combine_l2(outnum2, den2, skip2)
    return tc_pool(h2, batch)


def kernel(x_5x, edge_index_5x, batch_5x, x_10x, edge_index_10x, batch_10x,
           x_20x, edge_index_20x, batch_20x, params):
    p5 = _branch(x_5x, edge_index_5x, batch_5x, params["b5"])
    p10 = _branch(x_10x, edge_index_10x, batch_10x, params["b10"])
    p20 = _branch(x_20x, edge_index_20x, batch_20x, params["b20"])
    return tc_head(p5, p10, p20, params)


# speedup vs baseline: 12.3978x; 12.3978x over previous
import functools
import math

import jax
import jax.numpy as jnp
import numpy as np
from jax.experimental import pallas as pl
from jax.experimental.pallas import tpu as pltpu

N = 10000
E = 160000
F_IN = 256
HEADS = 8
C1 = 128
C2 = 8
D1 = HEADS * C1
D2 = HEADS * C2
NUM_GRAPHS = 16
NB = 400
NBLK = N // NB

_DOT = functools.partial(jax.lax.dot_general,
                         dimension_numbers=(((1,), (0,)), ((), ())),
                         preferred_element_type=jnp.float32,
                         precision=jax.lax.Precision.HIGHEST)


def _qkvs1_body(x_ref, wq_ref, wk_ref, wv_ref, ws_ref, bq_ref, bk_ref, bv_ref,
                bs_ref, q_ref, k_ref, v_ref, s_ref):
    x = x_ref[...]
    q_ref[0] = _DOT(x, wq_ref[...]) + bq_ref[0]
    k_ref[0] = _DOT(x, wk_ref[...]) + bk_ref[0]
    v_ref[0] = _DOT(x, wv_ref[...]) + bv_ref[0]
    s_ref[...] = _DOT(x, ws_ref[...]) + bs_ref[0]


def tc_qkvs_l1(x, cp):
    wq, wk, wv, ws = cp["q"]["W"], cp["k"]["W"], cp["v"]["W"], cp["skip"]["W"]
    bq = cp["q"]["b"].reshape(HEADS, 1, C1)
    bk = cp["k"]["b"].reshape(HEADS, 1, C1)
    bv = cp["v"]["b"].reshape(HEADS, 1, C1)
    bs = cp["skip"]["b"].reshape(HEADS, 1, C1)
    grid = (HEADS, NBLK)
    q, k, v, s = pl.pallas_call(
        _qkvs1_body,
        grid=grid,
        in_specs=[
            pl.BlockSpec((NB, F_IN), lambda h, nb: (nb, 0)),
            pl.BlockSpec((F_IN, C1), lambda h, nb: (0, h)),
            pl.BlockSpec((F_IN, C1), lambda h, nb: (0, h)),
            pl.BlockSpec((F_IN, C1), lambda h, nb: (0, h)),
            pl.BlockSpec((F_IN, C1), lambda h, nb: (0, h)),
            pl.BlockSpec((1, 1, C1), lambda h, nb: (h, 0, 0)),
            pl.BlockSpec((1, 1, C1), lambda h, nb: (h, 0, 0)),
            pl.BlockSpec((1, 1, C1), lambda h, nb: (h, 0, 0)),
            pl.BlockSpec((1, 1, C1), lambda h, nb: (h, 0, 0)),
        ],
        out_specs=[
            pl.BlockSpec((1, NB, C1), lambda h, nb: (h, nb, 0)),
            pl.BlockSpec((1, NB, C1), lambda h, nb: (h, nb, 0)),
            pl.BlockSpec((1, NB, C1), lambda h, nb: (h, nb, 0)),
            pl.BlockSpec((NB, C1), lambda h, nb: (nb, h)),
        ],
        out_shape=[
            jax.ShapeDtypeStruct((HEADS, N, C1), jnp.float32),
            jax.ShapeDtypeStruct((HEADS, N, C1), jnp.float32),
            jax.ShapeDtypeStruct((HEADS, N, C1), jnp.float32),
            jax.ShapeDtypeStruct((N, D1), jnp.float32),
        ],
    )(x, wq, wk, wv, ws, bq, bk, bv, bs)
    return q, k, v, s


def _qkvs2_body(x_ref, wq_ref, wkv_ref, bq_ref, bkv_ref, q_ref, kv_ref):
    x = x_ref[...]
    q_ref[...] = _DOT(x, wq_ref[...]) + bq_ref[0]
    kv_ref[...] = _DOT(x, wkv_ref[...]) + bkv_ref[0]


def tc_qkvs_l2(h1, cp, perm):
    wqp = jnp.concatenate([cp["q"]["W"][:, perm], cp["skip"]["W"]], axis=1)
    wkv = jnp.concatenate([cp["k"]["W"][:, perm], cp["v"]["W"]], axis=1)
    bqp = jnp.concatenate([cp["q"]["b"][perm], cp["skip"]["b"]]).reshape(1, 128)
    bkv = jnp.concatenate([cp["k"]["b"][perm], cp["v"]["b"]]).reshape(1, 128)
    return pl.pallas_call(
        _qkvs2_body,
        grid=(NBLK,),
        in_specs=[
            pl.BlockSpec((NB, D1), lambda nb: (nb, 0)),
            pl.BlockSpec((D1, 128), lambda nb: (0, 0)),
            pl.BlockSpec((D1, 128), lambda nb: (0, 0)),
            pl.BlockSpec((1, 128), lambda nb: (0, 0)),
            pl.BlockSpec((1, 128), lambda nb: (0, 0)),
        ],
        out_specs=[pl.BlockSpec((NB, 128), lambda nb: (nb, 0))] * 2,
        out_shape=[jax.ShapeDtypeStruct((N, 128), jnp.float32)] * 2,
    )(h1, wqp, wkv, bqp, bkv)


def _combine1_body(on_ref, dn_ref, sk_ref, h_ref):
    h = pl.program_id(0)
    dn = dn_ref[0] + dn_ref[1]
    lane = jax.lax.broadcasted_iota(jnp.int32, (1, 128), 1)
    den = jnp.sum(jnp.where(lane == h, dn, 0.0), axis=1, keepdims=True)
    h_ref[...] = jnp.maximum(on_ref[0] / (den + 1e-16) + sk_ref[...], 0.0)


def tc_combine_l1(outnum, denp, skip):
    return pl.pallas_call(
        _combine1_body,
        grid=(HEADS, NBLK),
        in_specs=[
            pl.BlockSpec((1, NB, C1), lambda h, nb: (h, nb, 0)),
            pl.BlockSpec((2, NB, 128), lambda h, nb: (0, nb, 0)),
            pl.BlockSpec((NB, C1), lambda h, nb: (nb, h)),
        ],
        out_specs=pl.BlockSpec((NB, C1), lambda h, nb: (nb, h)),
        out_shape=jax.ShapeDtypeStruct((N, D1), jnp.float32),
    )(outnum, denp, skip)


def _combine2_body(acc_ref, q2p_ref, h_ref):
    a = acc_ref[0] + acc_ref[1]
    q2p = q2p_ref[...]
    h_ref[...] = jnp.maximum(
        a[:, :D2] / (a[:, D2:] + 1e-16) + q2p[:, D2:], 0.0)


def tc_combine_l2(acc2, q2p):
    return pl.pallas_call(
        _combine2_body,
        grid=(NBLK,),
        in_specs=[
            pl.BlockSpec((2, NB, 128), lambda nb: (0, nb, 0)),
            pl.BlockSpec((NB, 128), lambda nb: (nb, 0)),
        ],
        out_specs=pl.BlockSpec((NB, D2), lambda nb: (nb, 0)),
        out_shape=jax.ShapeDtypeStruct((N, D2), jnp.float32),
    )(acc2, q2p)


def _pool_body(b_ref, h_ref, p_ref):
    nb = pl.program_id(0)

    @pl.when(nb == 0)
    def _():
        p_ref[...] = jnp.full((NUM_GRAPHS, D2), -jnp.inf, jnp.float32)

    b = b_ref[0]
    blk = h_ref[...]
    for g in range(NUM_GRAPHS):
        m = jnp.where(b == g, blk, -jnp.inf)
        p_ref[g, :] = jnp.maximum(p_ref[g, :], jnp.max(m, axis=0))


def tc_pool(h2, batch):
    b3 = batch.reshape(NBLK, NB, 1)
    return pl.pallas_call(
        _pool_body,
        grid=(NBLK,),
        in_specs=[
            pl.BlockSpec((1, NB, 1), lambda nb: (nb, 0, 0)),
            pl.BlockSpec((NB, D2), lambda nb: (nb, 0)),
        ],
        out_specs=pl.BlockSpec((NUM_GRAPHS, D2), lambda nb: (0, 0)),
        out_shape=jax.ShapeDtypeStruct((NUM_GRAPHS, D2), jnp.float32),
    )(b3, h2)


def _head_body(p5_ref, p10_ref, p20_ref, w5_ref, w10_ref, w20_ref, b3_ref,
               w1_ref, b1_ref, w2_ref, b2_ref, o_ref):
    outs = []
    for i, pref in enumerate((p5_ref, p10_ref, p20_ref)):
        p = pref[...]
        p = jnp.where(jnp.isfinite(p), p, 0.0)
        w = (w5_ref, w10_ref, w20_ref)[i][...]
        outs.append(_DOT(p, w) + b3_ref[i][None, :])
    x = jnp.concatenate(outs, axis=1)
    h = jnp.maximum(_DOT(x, w1_ref[...]) + b1_ref[0][None, :], 0.0)
    o_ref[...] = _DOT(h, w2_ref[...]) + b2_ref[0][None, :]


def tc_head(p5, p10, p20, params):
    b3 = jnp.stack([params["b5"]["mlp"]["b"], params["b10"]["mlp"]["b"],
                    params["b20"]["mlp"]["b"]])
    full = lambda *shape: pl.BlockSpec(shape, lambda: tuple(0 for _ in shape))
    return pl.pallas_call(
        _head_body,
        in_specs=[
            full(NUM_GRAPHS, D2), full(NUM_GRAPHS, D2), full(NUM_GRAPHS, D2),
            full(D2, 8), full(D2, 8), full(D2, 8), full(3, 8),
            full(24, 8), full(1, 8), full(8, 10), full(1, 10),
        ],
        out_specs=full(NUM_GRAPHS, 10),
        out_shape=jax.ShapeDtypeStruct((NUM_GRAPHS, 10), jnp.float32),
    )(p5, p10, p20, params["b5"]["mlp"]["W"], params["b10"]["mlp"]["W"],
      params["b20"]["mlp"]["W"], b3, params["fc1"]["W"],
      params["fc1"]["b"].reshape(1, 8), params["fc2"]["W"],
      params["fc2"]["b"].reshape(1, 10))


NP = 10240
NCORE = 2
NSUB = 16
NW = NCORE * NSUB
EW = E // NW
WB1 = 128
NWIN1 = -(-EW // WB1)
WC1 = 80
ET1 = E // NSUB
NROWS = NP // NSUB
HPC = HEADS // NCORE
CNT0 = (E // NW) // 16 * 16
WD = 64
INV_SQRT_C1 = 1.0 / math.sqrt(float(C1))
INV_SQRT_C2 = 1.0 / math.sqrt(float(C2))

from jax import lax
from jax.experimental.pallas import tpu_sc as plsc

_SC_MESH = dict(core_axis_name="c", subcore_axis_name="s")


def _sum16(acc):
    return ((((acc[0] + acc[1]) + (acc[2] + acc[3]))
             + ((acc[4] + acc[5]) + (acc[6] + acc[7])))
            + (((acc[8] + acc[9]) + (acc[10] + acc[11]))
               + ((acc[12] + acc[13]) + (acc[14] + acc[15]))))


def _fold_max(gbuf):
    m = gbuf[pl.ds(0, 16)]
    for jj in range(1, NW):
        m = jnp.maximum(m, gbuf[pl.ds(jj * 16, 16)])
    return m


def _zero_zbuf(zbuf):
    zv = jnp.zeros((16,), jnp.float32)

    def zrow(r, carry):
        for cb in range(8):
            zbuf[r, pl.ds(cb * 16, 16)] = zv
        return carry

    lax.fori_loop(0, 32, zrow, 0)


def _zero_acc(acc_sp, zbuf, s):
    def zc(r, carry):
        pltpu.sync_copy(zbuf, acc_sp.at[pl.ds(s * NROWS + r * 32, 32)])
        return carry

    lax.fori_loop(0, NROWS // 32, zc, 0)


def _alpha1_body(q_hbm, k_hbm, src_hbm, dst_hbm, alpha_hbm, gmax_hbm,
                 sidx, didx, gqA, gkA, gqB, gkB, qbufA, kbufA, qbufB, kbufB,
                 abuf, mbuf, tbufA, tbufB, semA, semB):
    c = lax.axis_index("c")
    s = lax.axis_index("s")
    w = s * NCORE + c
    base = w * EW
    lanes = lax.iota(jnp.int32, 16)
    lane7 = jnp.bitwise_and(lanes, 7)
    ninf = jnp.full((16,), -jnp.inf, jnp.float32)
    zv = jnp.zeros((16,), jnp.float32)
    tbufA[pl.ds(16, 16)] = zv
    tbufB[pl.ds(16, 16)] = zv

    def build(gq, gk, h):
        hv = jnp.full((16,), h * N, jnp.int32)

        def bidx(g, c4):
            sl = pl.ds(g * 16, 16)
            gq[sl] = didx[sl] + hv
            gk[sl] = sidx[sl] + hv
            return c4

        lax.fori_loop(0, WB1 // 16, bidx, 0)

    def fire(gq, gk, qb, kb, sem):
        pltpu.make_async_copy(q_hbm.at[gq], qb, sem).start()
        pltpu.make_async_copy(k_hbm.at[gk], kb, sem).start()

    def waitg(gq, gk, qb, kb, sem):
        pltpu.make_async_copy(q_hbm.at[gq], qb, sem).wait()
        pltpu.make_async_copy(k_hbm.at[gk], kb, sem).wait()

    def fold_sum(acc, tb):
        tb[pl.ds(0, 16)] = acc
        s1 = acc + tb[pl.ds(8, 16)]
        tb[pl.ds(0, 16)] = s1
        s2 = s1 + tb[pl.ds(4, 16)]
        tb[pl.ds(0, 16)] = s2
        s3 = s2 + tb[pl.ds(2, 16)]
        tb[pl.ds(0, 16)] = s3
        s4 = s3 + tb[pl.ds(1, 16)]
        return s4[0]

    def compute(qb, kb, h, mx):
        hmask = lane7 == h

        def ebody(e2, mx):
            for u, tb in ((0, tbufA), (1, tbufB)):
                e = 2 * e2 + u
                acc = qb[e, pl.ds(0, 16)] * kb[e, pl.ds(0, 16)]
                for cb in range(1, 8):
                    sl = pl.ds(cb * 16, 16)
                    acc = acc + qb[e, sl] * kb[e, sl]
                t = fold_sum(acc, tb)
                srow = jnp.full((16,), t * INV_SQRT_C1, jnp.float32)
                abuf[e, :] = jnp.where(hmask, srow, abuf[e, :])
                mx = jnp.maximum(mx, jnp.where(hmask, srow, ninf))
            return mx

        return lax.fori_loop(0, WB1 // 2, ebody, mx)

    sets = [(gqA, gkA, qbufA, kbufA, semA), (gqB, gkB, qbufB, kbufB, semB)]

    def win_body(i, mx):
        start = base + jnp.minimum(i * WB1, EW - WB1)
        pltpu.sync_copy(src_hbm.at[pl.ds(start, WB1)], sidx)
        pltpu.sync_copy(dst_hbm.at[pl.ds(start, WB1)], didx)
        build(gqA, gkA, 0)
        fire(*sets[0])
        for h in range(HEADS):
            cur = sets[h % 2]
            if h + 1 < HEADS:
                nxt = sets[(h + 1) % 2]
                build(nxt[0], nxt[1], h + 1)
                fire(*nxt)
            waitg(*cur)
            mx = compute(cur[2], cur[3], h, mx)
        pltpu.sync_copy(abuf, alpha_hbm.at[pl.ds(start, WB1), :])
        return mx

    mx = lax.fori_loop(0, NWIN1, win_body,
                       jnp.full((16,), -jnp.inf, jnp.float32))
    mbuf[...] = mx
    pltpu.sync_copy(mbuf, gmax_hbm.at[pl.ds(w * 16, 16)])


def _alpha1_call(qf, kf, src, dst):
    f = functools.partial(
        pl.kernel,
        out_type=[jax.ShapeDtypeStruct((E, 16), jnp.float32),
                  jax.ShapeDtypeStruct((NW * 16,), jnp.float32)],
        mesh=plsc.VectorSubcoreMesh(**_SC_MESH),
        scratch_types=[
            pltpu.VMEM((WB1,), jnp.int32), pltpu.VMEM((WB1,), jnp.int32),
            pltpu.VMEM((WB1,), jnp.int32), pltpu.VMEM((WB1,), jnp.int32),
            pltpu.VMEM((WB1,), jnp.int32), pltpu.VMEM((WB1,), jnp.int32),
            pltpu.VMEM((WB1, C1), jnp.float32),
            pltpu.VMEM((WB1, C1), jnp.float32),
            pltpu.VMEM((WB1, C1), jnp.float32),
            pltpu.VMEM((WB1, C1), jnp.float32),
            pltpu.VMEM((WB1, 16), jnp.float32),
            pltpu.VMEM((16,), jnp.float32),
            pltpu.VMEM((32,), jnp.float32), pltpu.VMEM((32,), jnp.float32),
            pltpu.SemaphoreType.DMA, pltpu.SemaphoreType.DMA,
        ],
    )(_alpha1_body)
    return f(qf, kf, src, dst)


def _agg1_body(v_hbm, alpha_hbm, gmax_hbm, pe_hbm,
               onum_hbm, den_hbm, acc_sp,
               sdbuf, didx1, didxA, gidxA, didxB, gidxB,
               abufA, abufB, vbufA, vbufB, gbuf, zbuf,
               semA, semB, semSA, semSB):
    c = lax.axis_index("c")
    s = lax.axis_index("s")
    w = s * NCORE + c
    _zero_zbuf(zbuf)
    pltpu.sync_copy(gmax_hbm, gbuf)
    m = _fold_max(gbuf)

    _zero_acc(acc_sp, zbuf, s)
    plsc.subcore_barrier()
    off = w * CNT0
    cnt = jnp.where(w == NW - 1, E - (NW - 1) * CNT0, CNT0)

    def dwin(i, carry):
        start = off + i * WD
        pltpu.sync_copy(pe_hbm.at[pl.ds(start, WD)], sdbuf.at[pl.ds(0, WD)])
        pltpu.sync_copy(alpha_hbm.at[pl.ds(start, WD), :],
                        abufA.at[pl.ds(0, WD), :])
        for g in range(WD // 16):
            sl = pl.ds(g * 16, 16)
            didx1[sl] = jnp.bitwise_and(sdbuf[sl], 16383)

        def dedge(e, c3):
            exv = jnp.exp(abufA[e] - m)
            for jj in range(8):
                vbufA[e, pl.ds(jj * 16, 16)] = exv
            return c3

        lax.fori_loop(0, WD, dedge, 0)
        pltpu.sync_copy(vbufA.at[pl.ds(0, WD), :], acc_sp.at[didx1],
                        add=True)
        return carry

    lax.fori_loop(0, cnt // WD, dwin, 0)
    plsc.subcore_barrier()
    pltpu.sync_copy(acc_sp.at[pl.ds(s * NROWS, NROWS)],
                    den_hbm.at[pl.ds(c * NP + s * NROWS, NROWS)])
    plsc.subcore_barrier()

    NWIN = ET1 // WC1
    PAIRS = NWIN // 2
    for cc in range(NCORE):
        @pl.when(c == cc)
        def _phase2():
            ebase = s * ET1
            for hi in range(HPC):
                h = cc * HPC + hi
                _zero_acc(acc_sp, zbuf, s)
                plsc.subcore_barrier()

                def load_win(i, didx, gidx):
                    start = ebase + i * WC1
                    pltpu.sync_copy(pe_hbm.at[pl.ds(start, WC1)], sdbuf)

                    def bidx(g, c4):
                        sl = pl.ds(g * 16, 16)
                        v = sdbuf[sl]
                        gidx[sl] = lax.shift_right_logical(v, 14) + h * N
                        didx[sl] = jnp.bitwise_and(v, 16383)
                        return c4

                    lax.fori_loop(0, WC1 // 16, bidx, 0)

                def fire(i, gidx, abuf, vbuf, sem):
                    start = ebase + i * WC1
                    pltpu.make_async_copy(v_hbm.at[gidx], vbuf, sem).start()
                    pltpu.make_async_copy(
                        alpha_hbm.at[pl.ds(start, WC1), :], abuf, sem).start()

                def wait(gidx, abuf, vbuf, sem):
                    pltpu.make_async_copy(v_hbm.at[gidx], vbuf, sem).wait()
                    pltpu.make_async_copy(
                        alpha_hbm.at[pl.ds(0, WC1), :], abuf, sem).wait()

                def process(abuf, vbuf):
                    def nedge(e, c4):
                        exv = jnp.exp(abuf[e] - m)
                        ex = jnp.full((16,), exv[h], jnp.float32)
                        for cb in range(8):
                            sl = pl.ds(cb * 16, 16)
                            vbuf[e, sl] = vbuf[e, sl] * ex
                        return c4

                    lax.fori_loop(0, WC1, nedge, 0, unroll=2)

                def fire_sc(vbuf, didx, semS):
                    pltpu.make_async_copy(
                        vbuf, acc_sp.at[didx], semS).start()

                def wait_sc(vbuf, didx, semS):
                    pltpu.make_async_copy(
                        vbuf, acc_sp.at[didx], semS).wait()

                load_win(0, didxA, gidxA)
                fire(0, gidxA, abufA, vbufA, semA)
                load_win(1, didxB, gidxB)
                fire(1, gidxB, abufB, vbufB, semB)

                def pair(p, carry):
                    t = 2 * p
                    wait(gidxA, abufA, vbufA, semA)
                    process(abufA, vbufA)
                    pltpu.sync_copy(vbufA, acc_sp.at[didxA], add=True)
                    load_win(t + 2, didxA, gidxA)
                    fire(t + 2, gidxA, abufA, vbufA, semA)
                    wait(gidxB, abufB, vbufB, semB)
                    process(abufB, vbufB)
                    pltpu.sync_copy(vbufB, acc_sp.at[didxB], add=True)

                    @pl.when(p < PAIRS - 1)
                    def _preB():
                        load_win(t + 3, didxB, gidxB)
                        fire(t + 3, gidxB, abufB, vbufB, semB)

                    return carry

                lax.fori_loop(0, PAIRS, pair, 0)
                wait(gidxA, abufA, vbufA, semA)
                process(abufA, vbufA)
                pltpu.sync_copy(vbufA, acc_sp.at[didxA], add=True)
                plsc.subcore_barrier()
                pltpu.sync_copy(
                    acc_sp.at[pl.ds(s * NROWS, NROWS)],
                    onum_hbm.at[pl.ds(h * NP + s * NROWS, NROWS)])
                plsc.subcore_barrier()


def _agg1_call(vf, alpha, gmax, pe):
    f = functools.partial(
        pl.kernel,
        out_type=[jax.ShapeDtypeStruct((HEADS * NP, C1), jnp.float32),
                  jax.ShapeDtypeStruct((NCORE * NP, 128), jnp.float32)],
        mesh=plsc.VectorSubcoreMesh(**_SC_MESH),
        scratch_types=[
            pltpu.VMEM_SHARED((NP, 128), jnp.float32),
            pltpu.VMEM((WC1,), jnp.int32), pltpu.VMEM((WD,), jnp.int32),
            pltpu.VMEM((WC1,), jnp.int32), pltpu.VMEM((WC1,), jnp.int32),
            pltpu.VMEM((WC1,), jnp.int32), pltpu.VMEM((WC1,), jnp.int32),
            pltpu.VMEM((WC1, 16), jnp.float32),
            pltpu.VMEM((WC1, 16), jnp.float32),
            pltpu.VMEM((WC1, C1), jnp.float32),
            pltpu.VMEM((WC1, C1), jnp.float32),
            pltpu.VMEM((NW * 16,), jnp.float32),
            pltpu.VMEM((32, 128), jnp.float32),
            pltpu.SemaphoreType.DMA, pltpu.SemaphoreType.DMA,
            pltpu.SemaphoreType.DMA, pltpu.SemaphoreType.DMA,
        ],
    )(_agg1_body)
    return f(vf, alpha, gmax, pe)


def _alpha2_body(q_hbm, kv_hbm, src_hbm, dst_hbm, alpha_hbm, gmax_hbm,
                 sidx, didx, qbuf, kbuf, abuf, mbuf, tbufA, tbufB):
    c = lax.axis_index("c")
    s = lax.axis_index("s")
    w = s * NCORE + c
    base = w * EW
    zv = jnp.zeros((16,), jnp.float32)
    tbufA[pl.ds(16, 16)] = zv
    tbufB[pl.ds(16, 16)] = zv

    def win_body(i, mx):
        start = base + jnp.minimum(i * WB1, EW - WB1)
        pltpu.sync_copy(src_hbm.at[pl.ds(start, WB1)], sidx)
        pltpu.sync_copy(dst_hbm.at[pl.ds(start, WB1)], didx)
        pltpu.sync_copy(q_hbm.at[didx], qbuf)
        pltpu.sync_copy(kv_hbm.at[sidx], kbuf)

        def edge_body(e2, mx):
            for u, tb in ((0, tbufA), (1, tbufB)):
                e = 2 * e2 + u
                acc = qbuf[e, pl.ds(0, 16)] * kbuf[e, pl.ds(0, 16)]
                for jb in range(1, 4):
                    sl = pl.ds(jb * 16, 16)
                    acc = acc + qbuf[e, sl] * kbuf[e, sl]
                tb[pl.ds(0, 16)] = acc
                a = (acc + tb[pl.ds(8, 16)]) * INV_SQRT_C2
                abuf[e, :] = a
                mx = jnp.maximum(mx, a)
            return mx

        mx = lax.fori_loop(0, WB1 // 2, edge_body, mx)
        pltpu.sync_copy(abuf, alpha_hbm.at[pl.ds(start, WB1), :])
        return mx

    mx = lax.fori_loop(0, NWIN1, win_body,
                       jnp.full((16,), -jnp.inf, jnp.float32))
    mbuf[...] = mx
    pltpu.sync_copy(mbuf, gmax_hbm.at[pl.ds(w * 16, 16)])


def _alpha2_call(q2p, kv2, src, dst):
    f = functools.partial(
        pl.kernel,
        out_type=[jax.ShapeDtypeStruct((E, 16), jnp.float32),
                  jax.ShapeDtypeStruct((NW * 16,), jnp.float32)],
        mesh=plsc.VectorSubcoreMesh(**_SC_MESH),
        scratch_types=[
            pltpu.VMEM((WB1,), jnp.int32), pltpu.VMEM((WB1,), jnp.int32),
            pltpu.VMEM((WB1, 128), jnp.float32),
            pltpu.VMEM((WB1, 128), jnp.float32),
            pltpu.VMEM((WB1, 16), jnp.float32),
            pltpu.VMEM((16,), jnp.float32),
            pltpu.VMEM((32,), jnp.float32), pltpu.VMEM((32,), jnp.float32),
        ],
    )(_alpha2_body)
    return f(q2p, kv2, src, dst)


def _agg2_body(kv_hbm, alpha_hbm, gmax_hbm, src_hbm, dst_hbm,
               acc_hbm, acc_sp,
               sidx, didx, abuf, kvbuf, wbuf, gbuf, zbuf):
    c = lax.axis_index("c")
    s = lax.axis_index("s")
    w = s * NCORE + c
    lanes = lax.iota(jnp.int32, 16)
    lo = lanes < 8
    _zero_zbuf(zbuf)
    _zero_acc(acc_sp, zbuf, s)
    plsc.subcore_barrier()
    pltpu.sync_copy(gmax_hbm, gbuf)
    m = _fold_max(gbuf)
    off = w * CNT0
    cnt = jnp.where(w == NW - 1, E - (NW - 1) * CNT0, CNT0)

    def win_body(i, carry):
        start = off + i * WD
        pltpu.sync_copy(src_hbm.at[pl.ds(start, WD)], sidx)
        pltpu.sync_copy(dst_hbm.at[pl.ds(start, WD)], didx)
        pltpu.sync_copy(alpha_hbm.at[pl.ds(start, WD), :], abuf)
        pltpu.sync_copy(kv_hbm.at[sidx], kvbuf)

        def edge_body(e, c3):
            exv = jnp.exp(abuf[e] - m)
            for j in range(4):
                mj = jnp.where(lo, jnp.full((16,), exv[2 * j], jnp.float32),
                               jnp.full((16,), exv[2 * j + 1], jnp.float32))
                sl = pl.ds(j * 16, 16)
                slv = pl.ds(64 + j * 16, 16)
                wbuf[e, sl] = kvbuf[e, slv] * mj
                wbuf[e, slv] = mj
            return c3

        lax.fori_loop(0, WD, edge_body, 0)
        pltpu.sync_copy(wbuf, acc_sp.at[didx], add=True)
        return carry

    lax.fori_loop(0, cnt // WD, win_body, 0)
    plsc.subcore_barrier()
    pltpu.sync_copy(acc_sp.at[pl.ds(s * NROWS, NROWS)],
                    acc_hbm.at[pl.ds(c * NP + s * NROWS, NROWS)])


def _agg2_call(kv2, alpha2, gmax2, src, dst):
    f = functools.partial(
        pl.kernel,
        out_type=[jax.ShapeDtypeStruct((NCORE * NP, 128), jnp.float32)],
        mesh=plsc.VectorSubcoreMesh(**_SC_MESH),
        scratch_types=[
            pltpu.VMEM_SHARED((NP, 128), jnp.float32),
            pltpu.VMEM((WD,), jnp.int32), pltpu.VMEM((WD,), jnp.int32),
            pltpu.VMEM((WD, 16), jnp.float32),
            pltpu.VMEM((WD, 128), jnp.float32),
            pltpu.VMEM((WD, 128), jnp.float32),
            pltpu.VMEM((NW * 16,), jnp.float32),
            pltpu.VMEM((32, 128), jnp.float32),
        ],
    )(_agg2_body)
    return f(kv2, alpha2, gmax2, src, dst)


def _edge_l1_sc(q, k, v, src, dst, pe):
    qf = q.reshape(HEADS * N, C1)
    kf = k.reshape(HEADS * N, C1)
    vf = v.reshape(HEADS * N, C1)
    alpha, gmax = _alpha1_call(qf, kf, src, dst)
    outnum, den = _agg1_call(vf, alpha, gmax, pe)
    return outnum.reshape(HEADS, NP, C1), den.reshape(NCORE, NP, 128)


def _edge_l2_sc(q2p, kv2, src, dst):
    alpha2, gmax2 = _alpha2_call(q2p, kv2, src, dst)
    (acc,) = _agg2_call(kv2, alpha2, gmax2, src, dst)
    return acc.reshape(NCORE, NP, 128)


_PERM = np.arange(D2).reshape(HEADS, C2).T.reshape(-1)


def _branch(x, edge_index, batch, bp):
    src, dst = edge_index[0], edge_index[1]
    pe = src * 16384 + dst
    q, k, v, skip = tc_qkvs_l1(x, bp["c1"])
    outnum, denp = _edge_l1_sc(q, k, v, src, dst, pe)
    h1 = tc_combine_l1(outnum, denp, skip)
    q2p, kv2 = tc_qkvs_l2(h1, bp["c2"], _PERM)
    acc2 = _edge_l2_sc(q2p, kv2, src, dst)
    h2 = tc_combine_l2(acc2, q2p)
    return tc_pool(h2, batch)


def kernel(x_5x, edge_index_5x, batch_5x, x_10x, edge_index_10x, batch_10x,
           x_20x, edge_index_20x, batch_20x, params):
    p5 = _branch(x_5x, edge_index_5x, batch_5x, params["b5"])
    p10 = _branch(x_10x, edge_index_10x, batch_10x, params["b10"])
    p20 = _branch(x_20x, edge_index_20x, batch_20x, params["b20"])
    return tc_head(p5, p10, p20, params)
